# Initial kernel scaffold; baseline (speedup 1.0000x reference)
import jax
import jax.numpy as jnp
from jax.experimental import pallas as pl


def kernel(X, scale, zero_point):
    raise NotImplementedError("write your pallas kernel here")


# speedup vs baseline: 436.5703x; 436.5703x over previous
import functools

import jax
import jax.numpy as jnp
from jax import lax
from jax.experimental import pallas as pl
from jax.experimental.pallas import tpu as pltpu
from jax.experimental.pallas import tpu_sc as plsc

ROWS = 12 * 2048
COLS = 2048
LANES = 16
NW = 32
ROWS_PER_W = ROWS // NW
R_CHUNK = 8
N_CHUNK = ROWS_PER_W // R_CHUNK
VECS = COLS // LANES

_mesh = plsc.VectorSubcoreMesh(core_axis_name="c", subcore_axis_name="s")


@functools.partial(
    pl.kernel,
    mesh=_mesh,
    out_type=jax.ShapeDtypeStruct((ROWS, COLS), jnp.float32),
    compiler_params=pltpu.CompilerParams(needs_layout_passes=False),
    scratch_types=[
        pltpu.VMEM((256,), jnp.float32),
        pltpu.VMEM((2, LANES), jnp.float32),
        pltpu.VMEM((R_CHUNK, COLS), jnp.float32),
        pltpu.VMEM((R_CHUNK, COLS), jnp.float32),
    ],
)
def _qsoftmax(x_hbm, table_hbm, ab_hbm, out_hbm, table_v, ab_v, in_v, out_v):
    wid = lax.axis_index("s") * 2 + lax.axis_index("c")
    pltpu.sync_copy(table_hbm, table_v)
    pltpu.sync_copy(ab_hbm, ab_v)
    a = ab_v[0, :]
    b = ab_v[1, :]

    def chunk_body(c, _):
        base = wid * ROWS_PER_W + c * R_CHUNK
        pltpu.sync_copy(x_hbm.at[pl.ds(base, R_CHUNK)], in_v)
        for r in range(R_CHUNK):
            def vec_body(v, acc):
                sl = pl.ds(v * LANES, LANES)
                t = in_v[r, sl] * a + b
                t = jnp.minimum(jnp.maximum(t, 0.5), 255.5)
                idx = t.astype(jnp.int32)
                y = plsc.load_gather(table_v, [idx])
                out_v[r, sl] = y
                return acc + y

            acc = lax.fori_loop(0, VECS, vec_body,
                                jnp.zeros((LANES,), jnp.float32))
            total = jnp.sum(acc)
            inv = 1.0 / jnp.full((LANES,), total, jnp.float32)

            def scale_body(v, carry):
                sl = pl.ds(v * LANES, LANES)
                out_v[r, sl] = out_v[r, sl] * inv
                return carry

            lax.fori_loop(0, VECS, scale_body, 0)
        pltpu.sync_copy(out_v, out_hbm.at[pl.ds(base, R_CHUNK)])
        return _

    lax.fori_loop(0, N_CHUNK, chunk_body, 0)


def kernel(X, scale, zero_point):
    s = scale[0]
    zp = zero_point[0]
    levels = jnp.arange(0, 256, dtype=jnp.float32)
    table = jnp.exp((levels - zp) * s)
    ab = jnp.stack([
        jnp.full((LANES,), 1.0 / s, jnp.float32),
        jnp.full((LANES,), zp + 0.5, jnp.float32),
    ])
    out = _qsoftmax(X.reshape(ROWS, COLS), table, ab)
    return out.reshape(X.shape)


# speedup vs baseline: 1678.1342x; 3.8439x over previous
import functools

import jax
import jax.numpy as jnp
from jax import lax
from jax.experimental import pallas as pl
from jax.experimental.pallas import tpu as pltpu
from jax.experimental.pallas import tpu_sc as plsc

ROWS = 12 * 2048
COLS = 2048
LANES = 16
NW = 32
ROWS_PER_W = ROWS // NW
R_CHUNK = 8
N_CHUNK = ROWS_PER_W // R_CHUNK
VECS = COLS // LANES

_mesh = plsc.VectorSubcoreMesh(core_axis_name="c", subcore_axis_name="s")


@functools.partial(
    pl.kernel,
    mesh=_mesh,
    out_type=jax.ShapeDtypeStruct((ROWS, COLS), jnp.float32),
    compiler_params=pltpu.CompilerParams(needs_layout_passes=False),
    scratch_types=[
        pltpu.VMEM((256,), jnp.float32),
        pltpu.VMEM((2, LANES), jnp.float32),
        pltpu.VMEM((2, R_CHUNK, COLS), jnp.float32),
        pltpu.VMEM((2, R_CHUNK, COLS), jnp.float32),
        pltpu.SemaphoreType.DMA((2,)),
        pltpu.SemaphoreType.DMA((2,)),
    ],
)
def _qsoftmax(x_hbm, table_hbm, ab_hbm, out_hbm, table_v, ab_v, in_v, out_v,
              in_sems, out_sems):
    wid = lax.axis_index("s") * 2 + lax.axis_index("c")
    row0 = wid * ROWS_PER_W
    pltpu.sync_copy(table_hbm, table_v)
    pltpu.sync_copy(ab_hbm, ab_v)
    a = ab_v[0, :]
    b = ab_v[1, :]

    def start_in(c, buf):
        pltpu.async_copy(x_hbm.at[pl.ds(row0 + c * R_CHUNK, R_CHUNK)],
                         in_v.at[buf], in_sems.at[buf])

    def wait_in(buf):
        pltpu.make_async_copy(x_hbm.at[pl.ds(0, R_CHUNK)], in_v.at[buf],
                              in_sems.at[buf]).wait()

    def start_out(c, buf):
        pltpu.async_copy(out_v.at[buf],
                         out_hbm.at[pl.ds(row0 + c * R_CHUNK, R_CHUNK)],
                         out_sems.at[buf])

    def wait_out(buf):
        pltpu.make_async_copy(out_v.at[buf], out_hbm.at[pl.ds(0, R_CHUNK)],
                              out_sems.at[buf]).wait()

    start_in(0, 0)

    def outer(g2, carry):
        for bfr in range(2):
            g = g2 * 2 + bfr

            @pl.when(g + 1 < N_CHUNK)
            def _prefetch():
                start_in(g + 1, 1 - bfr)

            wait_in(bfr)

            @pl.when(g >= 2)
            def _drain():
                wait_out(bfr)

            for r in range(R_CHUNK):
                def vec_body(v, acc):
                    sl = pl.ds(v * LANES, LANES)
                    t = in_v[bfr, r, sl] * a + b
                    t = jnp.minimum(jnp.maximum(t, 0.5), 255.5)
                    idx = t.astype(jnp.int32)
                    y = plsc.load_gather(table_v, [idx])
                    out_v[bfr, r, sl] = y
                    return acc + y

                acc = plsc.parallel_loop(
                    0, VECS, carry=jnp.zeros((LANES,), jnp.float32),
                    unroll=8)(vec_body)
                total = jnp.sum(acc)
                inv = 1.0 / jnp.full((LANES,), total, jnp.float32)

                def scale_body(v, c2):
                    sl = pl.ds(v * LANES, LANES)
                    out_v[bfr, r, sl] = out_v[bfr, r, sl] * inv
                    return c2

                plsc.parallel_loop(0, VECS, carry=jnp.int32(0),
                                   unroll=8)(scale_body)

            start_out(g, bfr)
        return carry

    lax.fori_loop(0, N_CHUNK // 2, outer, 0)
    wait_out(0)
    wait_out(1)


def kernel(X, scale, zero_point):
    s = scale[0]
    zp = zero_point[0]
    levels = jnp.arange(0, 256, dtype=jnp.float32)
    table = jnp.exp((levels - zp) * s)
    ab = jnp.stack([
        jnp.full((LANES,), 1.0 / s, jnp.float32),
        jnp.full((LANES,), zp + 0.5, jnp.float32),
    ])
    out = _qsoftmax(X.reshape(ROWS, COLS), table, ab)
    return out.reshape(X.shape)


# speedup vs baseline: 2002.2515x; 1.1931x over previous
import functools

import jax
import jax.numpy as jnp
from jax import lax
from jax.experimental import pallas as pl
from jax.experimental.pallas import tpu as pltpu
from jax.experimental.pallas import tpu_sc as plsc

ROWS = 12 * 2048
COLS = 2048
LANES = 16
NW = 32
ROWS_PER_W = ROWS // NW
R_CHUNK = 8
N_CHUNK = ROWS_PER_W // R_CHUNK
VECS = COLS // LANES

_MAGIC = jnp.float32(2.0 ** 23)
_MAGIC_HI = jnp.float32(2.0 ** 23 + 255.0)

_mesh = plsc.VectorSubcoreMesh(core_axis_name="c", subcore_axis_name="s")


@functools.partial(
    pl.kernel,
    mesh=_mesh,
    out_type=jax.ShapeDtypeStruct((ROWS, COLS), jnp.float32),
    compiler_params=pltpu.CompilerParams(needs_layout_passes=False),
    scratch_types=[
        pltpu.VMEM((256,), jnp.float32),
        pltpu.VMEM((2, LANES), jnp.float32),
        pltpu.VMEM((2, R_CHUNK, COLS), jnp.float32),
        pltpu.VMEM((2, R_CHUNK, COLS), jnp.float32),
        pltpu.SemaphoreType.DMA((2,)),
        pltpu.SemaphoreType.DMA((2,)),
    ],
)
def _qsoftmax(x_hbm, table_hbm, ab_hbm, out_hbm, table_v, ab_v, in_v, out_v,
              in_sems, out_sems):
    wid = lax.axis_index("s") * 2 + lax.axis_index("c")
    row0 = wid * ROWS_PER_W
    pltpu.sync_copy(table_hbm, table_v)
    pltpu.sync_copy(ab_hbm, ab_v)
    a = ab_v[0, :]
    b = ab_v[1, :]

    def start_in(c, buf):
        pltpu.async_copy(x_hbm.at[pl.ds(row0 + c * R_CHUNK, R_CHUNK)],
                         in_v.at[buf], in_sems.at[buf])

    def wait_in(buf):
        pltpu.make_async_copy(x_hbm.at[pl.ds(0, R_CHUNK)], in_v.at[buf],
                              in_sems.at[buf]).wait()

    def start_out(c, buf):
        pltpu.async_copy(out_v.at[buf],
                         out_hbm.at[pl.ds(row0 + c * R_CHUNK, R_CHUNK)],
                         out_sems.at[buf])

    def wait_out(buf):
        pltpu.make_async_copy(out_v.at[buf], out_hbm.at[pl.ds(0, R_CHUNK)],
                              out_sems.at[buf]).wait()

    start_in(0, 0)

    def outer(g2, carry):
        for bfr in range(2):
            g = g2 * 2 + bfr

            @pl.when(g + 1 < N_CHUNK)
            def _prefetch():
                start_in(g + 1, 1 - bfr)

            wait_in(bfr)

            @pl.when(g >= 2)
            def _drain():
                wait_out(bfr)

            for r in range(R_CHUNK):
                def vec_body(v, acc):
                    sl = pl.ds(v * LANES, LANES)
                    t = in_v[bfr, r, sl] * a + b
                    t = jnp.minimum(jnp.maximum(t, _MAGIC), _MAGIC_HI)
                    idx = plsc.bitcast(t, jnp.int32) & 0xFF
                    y = plsc.load_gather(table_v, [idx])
                    out_v[bfr, r, sl] = y
                    return acc + y

                acc = plsc.parallel_loop(
                    0, VECS, carry=jnp.zeros((LANES,), jnp.float32),
                    unroll=8)(vec_body)
                total = jnp.sum(acc)
                inv = 1.0 / jnp.full((LANES,), total, jnp.float32)

                def scale_body(v, c2):
                    sl = pl.ds(v * LANES, LANES)
                    out_v[bfr, r, sl] = out_v[bfr, r, sl] * inv
                    return c2

                plsc.parallel_loop(0, VECS, carry=jnp.int32(0),
                                   unroll=8)(scale_body)

            start_out(g, bfr)
        return carry

    lax.fori_loop(0, N_CHUNK // 2, outer, 0)
    wait_out(0)
    wait_out(1)


def kernel(X, scale, zero_point):
    s = scale[0]
    zp = zero_point[0]
    levels = jnp.arange(0, 256, dtype=jnp.float32)
    table = jnp.exp((levels - zp) * s)
    ab = jnp.stack([
        jnp.full((LANES,), 1.0 / s, jnp.float32),
        jnp.full((LANES,), zp + _MAGIC, jnp.float32),
    ])
    out = _qsoftmax(X.reshape(ROWS, COLS), table, ab)
    return out.reshape(X.shape)


# speedup vs baseline: 2750.0937x; 1.3735x over previous
import functools

import jax
import jax.numpy as jnp
from jax import lax
from jax.experimental import pallas as pl
from jax.experimental.pallas import tpu as pltpu
from jax.experimental.pallas import tpu_sc as plsc

ROWS = 12 * 2048
COLS = 2048
LANES = 16
NW = 32
RS_SC = 7680
ROWS_PER_W = RS_SC // NW
R_CHUNK = 8
N_CHUNK = ROWS_PER_W // R_CHUNK
VECS = COLS // LANES

RB_TC = 512
TC_BLK0 = RS_SC // RB_TC

_MAGIC = 2.0 ** 23
_MAGIC_HI = 2.0 ** 23 + 255.0

_mesh = plsc.VectorSubcoreMesh(core_axis_name="c", subcore_axis_name="s")


@functools.partial(
    pl.kernel,
    mesh=_mesh,
    out_type=jax.ShapeDtypeStruct((RS_SC, COLS), jnp.float32),
    compiler_params=pltpu.CompilerParams(needs_layout_passes=False),
    scratch_types=[
        pltpu.VMEM((256,), jnp.float32),
        pltpu.VMEM((2, LANES), jnp.float32),
        pltpu.VMEM((2, R_CHUNK, COLS), jnp.float32),
        pltpu.VMEM((2, R_CHUNK, COLS), jnp.float32),
        pltpu.SemaphoreType.DMA((2,)),
        pltpu.SemaphoreType.DMA((2,)),
    ],
)
def _qsoftmax_sc(x_hbm, table_hbm, ab_hbm, out_hbm, table_v, ab_v, in_v,
                 out_v, in_sems, out_sems):
    wid = lax.axis_index("s") * 2 + lax.axis_index("c")
    row0 = wid * ROWS_PER_W
    pltpu.sync_copy(table_hbm, table_v)
    pltpu.sync_copy(ab_hbm, ab_v)
    a = ab_v[0, :]
    b = ab_v[1, :]

    def start_in(c, buf):
        pltpu.async_copy(x_hbm.at[pl.ds(row0 + c * R_CHUNK, R_CHUNK)],
                         in_v.at[buf], in_sems.at[buf])

    def wait_in(buf):
        pltpu.make_async_copy(x_hbm.at[pl.ds(0, R_CHUNK)], in_v.at[buf],
                              in_sems.at[buf]).wait()

    def start_out(c, buf):
        pltpu.async_copy(out_v.at[buf],
                         out_hbm.at[pl.ds(row0 + c * R_CHUNK, R_CHUNK)],
                         out_sems.at[buf])

    def wait_out(buf):
        pltpu.make_async_copy(out_v.at[buf], out_hbm.at[pl.ds(0, R_CHUNK)],
                              out_sems.at[buf]).wait()

    start_in(0, 0)

    def outer(g2, carry):
        for bfr in range(2):
            g = g2 * 2 + bfr

            @pl.when(g + 1 < N_CHUNK)
            def _prefetch():
                start_in(g + 1, 1 - bfr)

            wait_in(bfr)

            @pl.when(g >= 2)
            def _drain():
                wait_out(bfr)

            for r in range(R_CHUNK):
                def vec_body(v, acc):
                    sl = pl.ds(v * LANES, LANES)
                    t = in_v[bfr, r, sl] * a + b
                    t = jnp.minimum(jnp.maximum(t, _MAGIC), _MAGIC_HI)
                    idx = plsc.bitcast(t, jnp.int32) & 0xFF
                    y = plsc.load_gather(table_v, [idx])
                    out_v[bfr, r, sl] = y
                    return acc + y

                acc = plsc.parallel_loop(
                    0, VECS, carry=jnp.zeros((LANES,), jnp.float32),
                    unroll=8)(vec_body)
                total = jnp.sum(acc)
                inv = 1.0 / jnp.full((LANES,), total, jnp.float32)

                def scale_body(v, c2):
                    sl = pl.ds(v * LANES, LANES)
                    out_v[bfr, r, sl] = out_v[bfr, r, sl] * inv
                    return c2

                plsc.parallel_loop(0, VECS, carry=jnp.int32(0),
                                   unroll=8)(scale_body)

            start_out(g, bfr)
        return carry

    lax.fori_loop(0, N_CHUNK // 2, outer, 0)
    wait_out(0)
    wait_out(1)


def _tc_body(ab_ref, x_ref, o_ref):
    a = ab_ref[0]
    c = ab_ref[1]
    s = ab_ref[2]
    t = x_ref[...] * a + c
    t = jnp.minimum(jnp.maximum(t, _MAGIC), _MAGIC_HI)
    y = jnp.exp((t - c) * s)
    denom = jnp.sum(y, axis=-1, keepdims=True)
    o_ref[...] = y * (1.0 / denom)


_tc_call = pl.pallas_call(
    _tc_body,
    grid=((ROWS - RS_SC) // RB_TC,),
    in_specs=[
        pl.BlockSpec(memory_space=pltpu.SMEM),
        pl.BlockSpec((RB_TC, COLS), lambda i: (i + TC_BLK0, 0)),
    ],
    out_specs=pl.BlockSpec((RB_TC, COLS), lambda i: (i + TC_BLK0, 0)),
    out_shape=jax.ShapeDtypeStruct((ROWS, COLS), jnp.float32),
)


def kernel(X, scale, zero_point):
    s = scale[0]
    zp = zero_point[0]
    levels = jnp.arange(0, 256, dtype=jnp.float32)
    table = jnp.exp((levels - zp) * s)
    ab = jnp.stack([
        jnp.full((LANES,), 1.0 / s, jnp.float32),
        jnp.full((LANES,), zp + _MAGIC, jnp.float32),
    ])
    ab_tc = jnp.stack([1.0 / s, zp + _MAGIC, s])
    x2 = X.reshape(ROWS, COLS)
    out_sc = _qsoftmax_sc(x2, table, ab)
    out_tc = _tc_call(ab_tc, x2)
    out = lax.dynamic_update_slice(out_tc, out_sc, (0, 0))
    return out.reshape(X.shape)


# speedup vs baseline: 2770.6415x; 1.0075x over previous
import functools

import jax
import jax.numpy as jnp
from jax import lax
from jax.experimental import pallas as pl
from jax.experimental.pallas import tpu as pltpu
from jax.experimental.pallas import tpu_sc as plsc

ROWS = 12 * 2048
COLS = 2048
LANES = 16
NW = 32
RS_SC = 7168
ROWS_PER_W = RS_SC // NW
R_CHUNK = 8
N_CHUNK = ROWS_PER_W // R_CHUNK
VECS = COLS // LANES

RB_TC = 512
TC_BLK0 = RS_SC // RB_TC

_MAGIC = 2.0 ** 23
_MAGIC_HI = 2.0 ** 23 + 255.0

_mesh = plsc.VectorSubcoreMesh(core_axis_name="c", subcore_axis_name="s")


@functools.partial(
    pl.kernel,
    mesh=_mesh,
    out_type=jax.ShapeDtypeStruct((RS_SC, COLS), jnp.float32),
    compiler_params=pltpu.CompilerParams(needs_layout_passes=False),
    scratch_types=[
        pltpu.VMEM((256,), jnp.float32),
        pltpu.VMEM((2, LANES), jnp.float32),
        pltpu.VMEM((2, R_CHUNK, COLS), jnp.float32),
        pltpu.VMEM((2, R_CHUNK, COLS), jnp.float32),
        pltpu.SemaphoreType.DMA((2,)),
        pltpu.SemaphoreType.DMA((2,)),
    ],
)
def _qsoftmax_sc(x_hbm, table_hbm, ab_hbm, out_hbm, table_v, ab_v, in_v,
                 out_v, in_sems, out_sems):
    wid = lax.axis_index("s") * 2 + lax.axis_index("c")
    row0 = wid * ROWS_PER_W
    pltpu.sync_copy(table_hbm, table_v)
    pltpu.sync_copy(ab_hbm, ab_v)
    a = ab_v[0, :]
    b = ab_v[1, :]

    def start_in(c, buf):
        pltpu.async_copy(x_hbm.at[pl.ds(row0 + c * R_CHUNK, R_CHUNK)],
                         in_v.at[buf], in_sems.at[buf])

    def wait_in(buf):
        pltpu.make_async_copy(x_hbm.at[pl.ds(0, R_CHUNK)], in_v.at[buf],
                              in_sems.at[buf]).wait()

    def start_out(c, buf):
        pltpu.async_copy(out_v.at[buf],
                         out_hbm.at[pl.ds(row0 + c * R_CHUNK, R_CHUNK)],
                         out_sems.at[buf])

    def wait_out(buf):
        pltpu.make_async_copy(out_v.at[buf], out_hbm.at[pl.ds(0, R_CHUNK)],
                              out_sems.at[buf]).wait()

    start_in(0, 0)

    def outer(g2, carry):
        for bfr in range(2):
            g = g2 * 2 + bfr

            @pl.when(g + 1 < N_CHUNK)
            def _prefetch():
                start_in(g + 1, 1 - bfr)

            wait_in(bfr)

            @pl.when(g >= 2)
            def _drain():
                wait_out(bfr)

            for r in range(R_CHUNK):
                def vec_body(v, acc):
                    sl = pl.ds(v * LANES, LANES)
                    t = in_v[bfr, r, sl] * a + b
                    t = jnp.minimum(jnp.maximum(t, _MAGIC), _MAGIC_HI)
                    idx = plsc.bitcast(t, jnp.int32) & 0xFF
                    y = plsc.load_gather(table_v, [idx])
                    out_v[bfr, r, sl] = y
                    return acc + y

                acc = plsc.parallel_loop(
                    0, VECS, carry=jnp.zeros((LANES,), jnp.float32),
                    unroll=8)(vec_body)
                total = jnp.sum(acc)
                inv = 1.0 / jnp.full((LANES,), total, jnp.float32)

                def scale_body(v, c2):
                    sl = pl.ds(v * LANES, LANES)
                    out_v[bfr, r, sl] = out_v[bfr, r, sl] * inv
                    return c2

                plsc.parallel_loop(0, VECS, carry=jnp.int32(0),
                                   unroll=8)(scale_body)

            start_out(g, bfr)
        return carry

    lax.fori_loop(0, N_CHUNK // 2, outer, 0)
    wait_out(0)
    wait_out(1)


def _tc_body(ab_ref, x_ref, o_ref):
    a = ab_ref[0]
    c = ab_ref[1]
    s = ab_ref[2]
    t = x_ref[...] * a + c
    t = jnp.minimum(jnp.maximum(t, _MAGIC), _MAGIC_HI)
    y = jnp.exp((t - c) * s)
    ones = jnp.ones((COLS, 1), jnp.float32)
    denom = jax.lax.dot_general(y, ones, (((1,), (0,)), ((), ())),
                                preferred_element_type=jnp.float32)
    o_ref[...] = y * (1.0 / denom)


_tc_call = pl.pallas_call(
    _tc_body,
    grid=((ROWS - RS_SC) // RB_TC,),
    in_specs=[
        pl.BlockSpec(memory_space=pltpu.SMEM),
        pl.BlockSpec((RB_TC, COLS), lambda i: (i + TC_BLK0, 0)),
    ],
    out_specs=pl.BlockSpec((RB_TC, COLS), lambda i: (i + TC_BLK0, 0)),
    out_shape=jax.ShapeDtypeStruct((ROWS, COLS), jnp.float32),
)


def kernel(X, scale, zero_point):
    s = scale[0]
    zp = zero_point[0]
    levels = jnp.arange(0, 256, dtype=jnp.float32)
    table = jnp.exp((levels - zp) * s)
    ab = jnp.stack([
        jnp.full((LANES,), 1.0 / s, jnp.float32),
        jnp.full((LANES,), zp + _MAGIC, jnp.float32),
    ])
    ab_tc = jnp.stack([1.0 / s, zp + _MAGIC, s])
    x2 = X.reshape(ROWS, COLS)
    out_sc = _qsoftmax_sc(x2, table, ab)
    out_tc = _tc_call(ab_tc, x2)
    out = lax.dynamic_update_slice(out_tc, out_sc, (0, 0))
    return out.reshape(X.shape)


# speedup vs baseline: 2863.0150x; 1.0333x over previous
import functools

import jax
import jax.numpy as jnp
from jax import lax
from jax.experimental import pallas as pl
from jax.experimental.pallas import tpu as pltpu
from jax.experimental.pallas import tpu_sc as plsc

ROWS = 12 * 2048
COLS = 2048
LANES = 16
NW = 32
RS_SC = 6144
ROWS_PER_W = RS_SC // NW
R_CHUNK = 8
N_CHUNK = ROWS_PER_W // R_CHUNK
VECS = COLS // LANES

RB_TC = 512
TC_BLK0 = RS_SC // RB_TC

_MAGIC = 2.0 ** 23
_MAGIC_HI = 2.0 ** 23 + 255.0

_mesh = plsc.VectorSubcoreMesh(core_axis_name="c", subcore_axis_name="s")


@functools.partial(
    pl.kernel,
    mesh=_mesh,
    out_type=jax.ShapeDtypeStruct((RS_SC, COLS), jnp.float32),
    compiler_params=pltpu.CompilerParams(needs_layout_passes=False),
    scratch_types=[
        pltpu.VMEM((256,), jnp.float32),
        pltpu.VMEM((2, LANES), jnp.float32),
        pltpu.VMEM((2, R_CHUNK, COLS), jnp.float32),
        pltpu.VMEM((2, R_CHUNK, COLS), jnp.float32),
        pltpu.SemaphoreType.DMA((2,)),
        pltpu.SemaphoreType.DMA((2,)),
    ],
)
def _qsoftmax_sc(x_hbm, table_hbm, ab_hbm, out_hbm, table_v, ab_v, in_v,
                 out_v, in_sems, out_sems):
    wid = lax.axis_index("s") * 2 + lax.axis_index("c")
    row0 = wid * ROWS_PER_W
    pltpu.sync_copy(table_hbm, table_v)
    pltpu.sync_copy(ab_hbm, ab_v)
    a = ab_v[0, :]
    b = ab_v[1, :]

    def start_in(c, buf):
        pltpu.async_copy(x_hbm.at[pl.ds(row0 + c * R_CHUNK, R_CHUNK)],
                         in_v.at[buf], in_sems.at[buf])

    def wait_in(buf):
        pltpu.make_async_copy(x_hbm.at[pl.ds(0, R_CHUNK)], in_v.at[buf],
                              in_sems.at[buf]).wait()

    def start_out(c, buf):
        pltpu.async_copy(out_v.at[buf],
                         out_hbm.at[pl.ds(row0 + c * R_CHUNK, R_CHUNK)],
                         out_sems.at[buf])

    def wait_out(buf):
        pltpu.make_async_copy(out_v.at[buf], out_hbm.at[pl.ds(0, R_CHUNK)],
                              out_sems.at[buf]).wait()

    start_in(0, 0)

    def outer(g2, carry):
        for bfr in range(2):
            g = g2 * 2 + bfr

            @pl.when(g + 1 < N_CHUNK)
            def _prefetch():
                start_in(g + 1, 1 - bfr)

            wait_in(bfr)

            @pl.when(g >= 2)
            def _drain():
                wait_out(bfr)

            for r in range(R_CHUNK):
                def vec_body(v, acc):
                    sl = pl.ds(v * LANES, LANES)
                    t = in_v[bfr, r, sl] * a + b
                    t = jnp.minimum(jnp.maximum(t, _MAGIC), _MAGIC_HI)
                    idx = plsc.bitcast(t, jnp.int32) & 0xFF
                    y = plsc.load_gather(table_v, [idx])
                    out_v[bfr, r, sl] = y
                    return acc + y

                acc = plsc.parallel_loop(
                    0, VECS, carry=jnp.zeros((LANES,), jnp.float32),
                    unroll=8)(vec_body)
                total = jnp.sum(acc)
                inv = 1.0 / jnp.full((LANES,), total, jnp.float32)

                def scale_body(v, c2):
                    sl = pl.ds(v * LANES, LANES)
                    out_v[bfr, r, sl] = out_v[bfr, r, sl] * inv
                    return c2

                plsc.parallel_loop(0, VECS, carry=jnp.int32(0),
                                   unroll=8)(scale_body)

            start_out(g, bfr)
        return carry

    lax.fori_loop(0, N_CHUNK // 2, outer, 0)
    wait_out(0)
    wait_out(1)


def _tc_body(ab_ref, x_ref, o_ref):
    a = ab_ref[0]
    c = ab_ref[1]
    s = ab_ref[2]
    t = x_ref[...] * a + c
    t = jnp.minimum(jnp.maximum(t, _MAGIC), _MAGIC_HI)
    y = jnp.exp((t - c) * s)
    ones = jnp.ones((COLS, 1), jnp.float32)
    denom = jax.lax.dot_general(y, ones, (((1,), (0,)), ((), ())),
                                preferred_element_type=jnp.float32)
    o_ref[...] = y * (1.0 / denom)


_tc_call = pl.pallas_call(
    _tc_body,
    grid=((ROWS - RS_SC) // RB_TC,),
    in_specs=[
        pl.BlockSpec(memory_space=pltpu.SMEM),
        pl.BlockSpec((RB_TC, COLS), lambda i: (i + TC_BLK0, 0)),
    ],
    out_specs=pl.BlockSpec((RB_TC, COLS), lambda i: (i + TC_BLK0, 0)),
    out_shape=jax.ShapeDtypeStruct((ROWS, COLS), jnp.float32),
)


def kernel(X, scale, zero_point):
    s = scale[0]
    zp = zero_point[0]
    levels = jnp.arange(0, 256, dtype=jnp.float32)
    table = jnp.exp((levels - zp) * s)
    ab = jnp.stack([
        jnp.full((LANES,), 1.0 / s, jnp.float32),
        jnp.full((LANES,), zp + _MAGIC, jnp.float32),
    ])
    ab_tc = jnp.stack([1.0 / s, zp + _MAGIC, s])
    x2 = X.reshape(ROWS, COLS)
    out_sc = _qsoftmax_sc(x2, table, ab)
    out_tc = _tc_call(ab_tc, x2)
    out = lax.dynamic_update_slice(out_tc, out_sc, (0, 0))
    return out.reshape(X.shape)


# speedup vs baseline: 2930.8962x; 1.0237x over previous
import functools

import jax
import jax.numpy as jnp
from jax import lax
from jax.experimental import pallas as pl
from jax.experimental.pallas import tpu as pltpu
from jax.experimental.pallas import tpu_sc as plsc

ROWS = 12 * 2048
COLS = 2048
LANES = 16
NW = 32
RS_SC = 6144
ROWS_PER_W = RS_SC // NW
R_CHUNK = 8
N_CHUNK = ROWS_PER_W // R_CHUNK
VECS = COLS // LANES

RB_TC = 1024
TC_BLK0 = RS_SC // RB_TC

_MAGIC = 2.0 ** 23
_MAGIC_HI = 2.0 ** 23 + 255.0

_mesh = plsc.VectorSubcoreMesh(core_axis_name="c", subcore_axis_name="s")


@functools.partial(
    pl.kernel,
    mesh=_mesh,
    out_type=jax.ShapeDtypeStruct((RS_SC, COLS), jnp.float32),
    compiler_params=pltpu.CompilerParams(needs_layout_passes=False),
    scratch_types=[
        pltpu.VMEM((256,), jnp.float32),
        pltpu.VMEM((2, LANES), jnp.float32),
        pltpu.VMEM((2, R_CHUNK, COLS), jnp.float32),
        pltpu.VMEM((2, R_CHUNK, COLS), jnp.float32),
        pltpu.SemaphoreType.DMA((2,)),
        pltpu.SemaphoreType.DMA((2,)),
    ],
)
def _qsoftmax_sc(x_hbm, table_hbm, ab_hbm, out_hbm, table_v, ab_v, in_v,
                 out_v, in_sems, out_sems):
    wid = lax.axis_index("s") * 2 + lax.axis_index("c")
    row0 = wid * ROWS_PER_W
    pltpu.sync_copy(table_hbm, table_v)
    pltpu.sync_copy(ab_hbm, ab_v)
    a = ab_v[0, :]
    b = ab_v[1, :]

    def start_in(c, buf):
        pltpu.async_copy(x_hbm.at[pl.ds(row0 + c * R_CHUNK, R_CHUNK)],
                         in_v.at[buf], in_sems.at[buf])

    def wait_in(buf):
        pltpu.make_async_copy(x_hbm.at[pl.ds(0, R_CHUNK)], in_v.at[buf],
                              in_sems.at[buf]).wait()

    def start_out(c, buf):
        pltpu.async_copy(out_v.at[buf],
                         out_hbm.at[pl.ds(row0 + c * R_CHUNK, R_CHUNK)],
                         out_sems.at[buf])

    def wait_out(buf):
        pltpu.make_async_copy(out_v.at[buf], out_hbm.at[pl.ds(0, R_CHUNK)],
                              out_sems.at[buf]).wait()

    start_in(0, 0)

    def outer(g2, carry):
        for bfr in range(2):
            g = g2 * 2 + bfr

            @pl.when(g + 1 < N_CHUNK)
            def _prefetch():
                start_in(g + 1, 1 - bfr)

            wait_in(bfr)

            @pl.when(g >= 2)
            def _drain():
                wait_out(bfr)

            for r in range(R_CHUNK):
                def vec_body(v, acc):
                    sl = pl.ds(v * LANES, LANES)
                    t = in_v[bfr, r, sl] * a + b
                    t = jnp.minimum(jnp.maximum(t, _MAGIC), _MAGIC_HI)
                    idx = plsc.bitcast(t, jnp.int32) & 0xFF
                    y = plsc.load_gather(table_v, [idx])
                    out_v[bfr, r, sl] = y
                    return acc + y

                acc = plsc.parallel_loop(
                    0, VECS, carry=jnp.zeros((LANES,), jnp.float32),
                    unroll=8)(vec_body)
                total = jnp.sum(acc)
                inv = 1.0 / jnp.full((LANES,), total, jnp.float32)

                def scale_body(v, c2):
                    sl = pl.ds(v * LANES, LANES)
                    out_v[bfr, r, sl] = out_v[bfr, r, sl] * inv
                    return c2

                plsc.parallel_loop(0, VECS, carry=jnp.int32(0),
                                   unroll=8)(scale_body)

            start_out(g, bfr)
        return carry

    lax.fori_loop(0, N_CHUNK // 2, outer, 0)
    wait_out(0)
    wait_out(1)


def _tc_body(ab_ref, x_ref, o_ref):
    a = ab_ref[0]
    c = ab_ref[1]
    s = ab_ref[2]
    t = x_ref[...] * a + c
    t = jnp.minimum(jnp.maximum(t, _MAGIC), _MAGIC_HI)
    y = jnp.exp((t - c) * s)
    ones = jnp.ones((COLS, 1), jnp.float32)
    denom = jax.lax.dot_general(y, ones, (((1,), (0,)), ((), ())),
                                preferred_element_type=jnp.float32)
    o_ref[...] = y * (1.0 / denom)


_tc_call = pl.pallas_call(
    _tc_body,
    grid=((ROWS - RS_SC) // RB_TC,),
    in_specs=[
        pl.BlockSpec(memory_space=pltpu.SMEM),
        pl.BlockSpec((RB_TC, COLS), lambda i: (i + TC_BLK0, 0)),
    ],
    out_specs=pl.BlockSpec((RB_TC, COLS), lambda i: (i + TC_BLK0, 0)),
    out_shape=jax.ShapeDtypeStruct((ROWS, COLS), jnp.float32),
)


def kernel(X, scale, zero_point):
    s = scale[0]
    zp = zero_point[0]
    levels = jnp.arange(0, 256, dtype=jnp.float32)
    table = jnp.exp((levels - zp) * s)
    ab = jnp.stack([
        jnp.full((LANES,), 1.0 / s, jnp.float32),
        jnp.full((LANES,), zp + _MAGIC, jnp.float32),
    ])
    ab_tc = jnp.stack([1.0 / s, zp + _MAGIC, s])
    x2 = X.reshape(ROWS, COLS)
    out_sc = _qsoftmax_sc(x2, table, ab)
    out_tc = _tc_call(ab_tc, x2)
    out = lax.dynamic_update_slice(out_tc, out_sc, (0, 0))
    return out.reshape(X.shape)
